# baseline (speedup 1.0000x reference)
import functools

import jax
import jax.numpy as jnp
from jax import lax
from jax.experimental import pallas as pl
from jax.experimental.pallas import tpu as pltpu
from jax.experimental.pallas import tpu_sc as plsc

K = 64
N = 8192
L = 16
R_TOTAL = 8192
NW = 32
ROWS_PER_W = R_TOTAL // NW
BATCH = 8
NGRP = 32


def _sort16(v, desc):
    if desc:
        return plsc.sort_key_val(v, v, descending=True)[0]
    return jnp.sort(v)


def _sort_bitonic_dir(vs, desc):
    if len(vs) == 1:
        return [_sort16(vs[0], desc)]
    h = len(vs) // 2
    lo = [jnp.minimum(a, b) for a, b in zip(vs[:h], vs[h:])]
    hi = [jnp.maximum(a, b) for a, b in zip(vs[:h], vs[h:])]
    if desc:
        return _sort_bitonic_dir(hi, True) + _sort_bitonic_dir(lo, True)
    return _sort_bitonic_dir(lo, False) + _sort_bitonic_dir(hi, False)


def _merge_dir(A, B, desc, cap):
    hi = [jnp.maximum(a, b) for a, b in zip(A, B)]
    if cap:
        return _sort_bitonic_dir(hi, desc)
    lo = [jnp.minimum(a, b) for a, b in zip(A, B)]
    if desc:
        return _sort_bitonic_dir(hi, True) + _sort_bitonic_dir(lo, True)
    return _sort_bitonic_dir(lo, False) + _sort_bitonic_dir(hi, False)


def _tree_topk(leaf_fn, n, desc):
    def build(idx, count, desc):
        if count == 1:
            return [_sort16(leaf_fn(idx), desc)]
        h = count // 2
        A = build(idx, h, False)
        B = build(idx + h, h, True)
        return _merge_dir(A, B, desc, cap=(len(A) == 4))
    return build(0, n, desc)


def _kv_exchange(ak, av, bk, bv):
    m = ak <= bk
    lok = jnp.minimum(ak, bk)
    hik = jnp.maximum(ak, bk)
    lov = jnp.where(m, av, bv)
    hiv = jnp.where(m, bv, av)
    return lok, lov, hik, hiv


def _kv_sort_bitonic_dir(ks, vs, desc):
    if len(ks) == 1:
        sk, sv = plsc.sort_key_val(ks[0], vs[0], descending=desc)
        return [sk], [sv]
    h = len(ks) // 2
    lok, lov, hik, hiv = [], [], [], []
    for a, va, b, vb in zip(ks[:h], vs[:h], ks[h:], vs[h:]):
        lk, lv, hk, hv = _kv_exchange(a, va, b, vb)
        lok.append(lk)
        lov.append(lv)
        hik.append(hk)
        hiv.append(hv)
    if desc:
        k1, v1 = _kv_sort_bitonic_dir(hik, hiv, True)
        k2, v2 = _kv_sort_bitonic_dir(lok, lov, True)
        return k1 + k2, v1 + v2
    k1, v1 = _kv_sort_bitonic_dir(lok, lov, False)
    k2, v2 = _kv_sort_bitonic_dir(hik, hiv, False)
    return k1 + k2, v1 + v2


def _kv_merge_dir(Ak, Av, Bk, Bv, desc, cap):
    hik, hiv, lok, lov = [], [], [], []
    for a, va, b, vb in zip(Ak, Av, Bk, Bv):
        lk, lv, hk, hv = _kv_exchange(a, va, b, vb)
        hik.append(hk)
        hiv.append(hv)
        lok.append(lk)
        lov.append(lv)
    if cap:
        return _kv_sort_bitonic_dir(hik, hiv, desc)
    if desc:
        k1, v1 = _kv_sort_bitonic_dir(hik, hiv, True)
        k2, v2 = _kv_sort_bitonic_dir(lok, lov, True)
        return k1 + k2, v1 + v2
    k1, v1 = _kv_sort_bitonic_dir(lok, lov, False)
    k2, v2 = _kv_sort_bitonic_dir(hik, hiv, False)
    return k1 + k2, v1 + v2


def _kv_tree_topk(leaf_fn, n, desc):
    def build(idx, count, desc):
        if count == 1:
            k, v = leaf_fn(idx)
            sk, sv = plsc.sort_key_val(k, v, descending=desc)
            return [sk], [sv]
        h = count // 2
        Ak, Av = build(idx, h, False)
        Bk, Bv = build(idx + h, h, True)
        return _kv_merge_dir(Ak, Av, Bk, Bv, desc, cap=(len(Ak) == 4))
    return build(0, n, desc)


def _process_row(row_v, out_v, off, r):
    iota = lax.iota(jnp.int32, L)

    def cm_leaf(g):
        vs = [row_v[pl.ds(off + g * 256 + j * L, L)] for j in range(16)]
        while len(vs) > 1:
            vs = [jnp.maximum(a, b) for a, b in zip(vs[0::2], vs[1::2])]
        return vs[0], g * 256 + iota

    _, vals4 = _kv_tree_topk(cm_leaf, NGRP, desc=False)
    bases = [v + off for v in vals4]

    def col_leaf(idx):
        return plsc.load_gather(row_v, [bases[idx // 16] + (idx % 16) * L])

    top = _tree_topk(col_leaf, 64, desc=True)
    for j in range(4):
        out_v[pl.ds(r * K + j * L, L)] = top[j]


def _sc_topk(x_hbm, out_hbm, row_v, out_v, sem):
    wid = lax.axis_index("s") * 2 + lax.axis_index("c")
    base = wid * ROWS_PER_W

    def batch_body(b, _):
        rows0 = base + b * BATCH
        copy = pltpu.make_async_copy(
            x_hbm.at[pl.ds(rows0 * N, BATCH * N)], row_v, sem)
        copy.start()
        copy.wait()

        def row_body(i, _):
            _process_row(row_v, out_v, i * N, b * BATCH + i)
            return 0

        lax.fori_loop(0, BATCH, row_body, 0, unroll=False)
        return 0

    lax.fori_loop(0, ROWS_PER_W // BATCH, batch_body, 0, unroll=False)

    out_copy = pltpu.make_async_copy(
        out_v, out_hbm.at[pl.ds(base * K, ROWS_PER_W * K)], sem)
    out_copy.start()
    out_copy.wait()


@jax.jit
def kernel(x):
    B, S, _ = x.shape
    mesh = plsc.VectorSubcoreMesh(core_axis_name="c", subcore_axis_name="s")
    run = pl.kernel(
        _sc_topk,
        out_type=jax.ShapeDtypeStruct((R_TOTAL * K,), jnp.float32),
        mesh=mesh,
        compiler_params=pltpu.CompilerParams(needs_layout_passes=False),
        scratch_types=[
            pltpu.VMEM((BATCH * N,), jnp.float32),
            pltpu.VMEM((ROWS_PER_W * K,), jnp.float32),
            pltpu.SemaphoreType.DMA,
        ],
    )
    out = run(x.reshape(R_TOTAL * N))
    return out.reshape(B, S, K)
